# Initial kernel scaffold; baseline (speedup 1.0000x reference)
import jax
import jax.numpy as jnp
from jax.experimental import pallas as pl


def kernel(time_wordcount, beta, topic_embeddings, word_embeddings):
    raise NotImplementedError("write your pallas kernel here")


# speedup vs baseline: 12.6412x; 12.6412x over previous
import jax
import jax.numpy as jnp
from jax.experimental import pallas as pl
from jax.experimental.pallas import tpu as pltpu

_T, _K, _V, _E = 128, 32, 8192, 16
_TEMP = 0.07
_NEG = 32


def _body(tw_ref, beta_ref, temb_ref, wemb_ref, out_ref, acc_ref):
    t = pl.program_id(0)
    X = beta_ref[0]
    tw = tw_ref[0]

    Xm = X
    cum = jnp.zeros((_K, 1), jnp.float32)
    tau = jnp.full((_K, 1), -jnp.inf, jnp.float32)
    for _ in range(_NEG):
        m = jnp.max(Xm, axis=1, keepdims=True)
        hit = Xm == m
        c = jnp.sum(hit.astype(jnp.float32), axis=1, keepdims=True)
        pick = cum < _NEG
        tau = jnp.where(pick, m, tau)
        cum = cum + jnp.where(pick, c, 0.0)
        Xm = jnp.where(hit, -jnp.inf, Xm)
    member = jnp.any(X >= tau, axis=0, keepdims=True)
    negm = member & (tw == 0)

    a = temb_ref[0]
    a = a / (jnp.sqrt(jnp.sum(a * a, axis=-1, keepdims=True)) + 1e-12)
    b = wemb_ref[...]
    b = b / (jnp.sqrt(jnp.sum(b * b, axis=-1, keepdims=True)) + 1e-12)
    sim = jax.lax.dot_general(
        a, b, (((1,), (1,)), ((), ())),
        preferred_element_type=jnp.float32) / _TEMP

    sim_m = jnp.where(negm, sim, -1e9)
    m = jnp.max(sim_m, axis=1, keepdims=True)
    lse = jnp.log(jnp.sum(jnp.exp(sim_m - m), axis=1, keepdims=True)) + m
    loss_t = jnp.sum(lse) / _K
    valid = jnp.any(negm)

    @pl.when(t == 0)
    def _init():
        acc_ref[0] = 0.0
        acc_ref[1] = 0.0

    acc_ref[0] += jnp.where(valid, loss_t, 0.0)
    acc_ref[1] += valid.astype(jnp.float32)

    @pl.when(t == _T - 1)
    def _fin():
        cnt = acc_ref[1]
        out_ref[0, 0] = jnp.where(
            cnt > 0.0, acc_ref[0] / jnp.maximum(cnt, 1.0), 0.0)


def kernel(time_wordcount, beta, topic_embeddings, word_embeddings):
    tw3 = time_wordcount.reshape(_T, 1, _V)
    out = pl.pallas_call(
        _body,
        grid=(_T,),
        in_specs=[
            pl.BlockSpec((1, 1, _V), lambda t: (t, 0, 0)),
            pl.BlockSpec((1, _K, _V), lambda t: (t, 0, 0)),
            pl.BlockSpec((1, _K, _E), lambda t: (t, 0, 0)),
            pl.BlockSpec((_V, _E), lambda t: (0, 0)),
        ],
        out_specs=pl.BlockSpec(memory_space=pltpu.SMEM),
        out_shape=jax.ShapeDtypeStruct((1, 1), jnp.float32),
        scratch_shapes=[pltpu.SMEM((2,), jnp.float32)],
        compiler_params=pltpu.CompilerParams(
            dimension_semantics=("arbitrary",)),
    )(tw3, beta, topic_embeddings, word_embeddings)
    return out[0, 0]
